# baseline (speedup 1.0000x reference)
import functools

import jax
import jax.numpy as jnp
from jax import lax
from jax.experimental import pallas as pl
from jax.experimental.pallas import tpu as pltpu
from jax.experimental.pallas import tpu_sc as plsc

N = 10000
E = 160000
H = 256
HH = 128
L = 3
NSP = 100
CUTOFF = 6.0

BN = 2000
BE = 2000

NS = 16
EPT = E // NS
CH = 80
NCHUNK = EPT // CH
ZR = 40
NZCH = N // ZR
ZPT = -(-NZCH // NS)
_F32 = jnp.float32


def _mlp(x, w1, b1, w2, b2):
    t = jax.nn.silu(jnp.dot(x, w1, preferred_element_type=_F32) + b1)
    return jnp.dot(t, w2, preferred_element_type=_F32) + b2


def _dist_body(d_ref, w1_ref, b1_ref, w2_ref, b2_ref, oa_ref, ob_ref):
    d = d_ref[...]
    env = 0.5 * (jnp.cos((jnp.pi / CUTOFF) * d) + 1.0)
    t = jax.nn.silu(d * w1_ref[...] + b1_ref[...])
    x = jnp.dot(t, w2_ref[...], preferred_element_type=_F32) + b2_ref[...]
    x = x * env
    xr = x.reshape(BE // 2, 2, H)
    lo = lax.bitcast_convert_type(xr[:, 0, :].astype(jnp.bfloat16),
                                  jnp.uint16).astype(jnp.uint32)
    hi = lax.bitcast_convert_type(xr[:, 1, :].astype(jnp.bfloat16),
                                  jnp.uint16).astype(jnp.uint32)
    w = lax.bitcast_convert_type(lo | (hi << 16), jnp.int32)
    oa_ref[...] = w[:, :HH]
    ob_ref[...] = w[:, HH:]


def _dist_call(d2, w1, b1, w2, b2):
    return pl.pallas_call(
        _dist_body,
        grid=(E // BE,),
        in_specs=[
            pl.BlockSpec((BE, 1), lambda i: (i, 0)),
            pl.BlockSpec((1, H), lambda i: (0, 0)),
            pl.BlockSpec((1, H), lambda i: (0, 0)),
            pl.BlockSpec((H, H), lambda i: (0, 0)),
            pl.BlockSpec((1, H), lambda i: (0, 0)),
        ],
        out_specs=[
            pl.BlockSpec((BE // 2, HH), lambda i: (i, 0)),
            pl.BlockSpec((BE // 2, HH), lambda i: (i, 0)),
        ],
        out_shape=[jax.ShapeDtypeStruct((E // 2, HH), jnp.int32)] * 2,
    )(d2, w1, b1, w2, b2)


def _k0_body(sp_ref, emb_ref, w1_ref, b1_ref, w2_ref, b2_ref,
             h_ref, ma_ref, mb_ref):
    sp = sp_ref[0, 0]
    oh = (sp[:, None] == lax.broadcasted_iota(jnp.int32, (BN, 128), 1))
    h0 = jnp.dot(oh.astype(_F32), emb_ref[...], preferred_element_type=_F32)
    h_ref[...] = h0
    m = _mlp(h0, w1_ref[...], b1_ref[...], w2_ref[...], b2_ref[...])
    ma_ref[...] = m[:, :HH]
    mb_ref[...] = m[:, HH:]


def _k0_call(sp3, emb_p, w1, b1, w2, b2):
    return pl.pallas_call(
        _k0_body,
        grid=(N // BN,),
        in_specs=[
            pl.BlockSpec((1, 1, BN), lambda i: (i, 0, 0)),
            pl.BlockSpec((128, H), lambda i: (0, 0)),
            pl.BlockSpec((H, H), lambda i: (0, 0)),
            pl.BlockSpec((1, H), lambda i: (0, 0)),
            pl.BlockSpec((H, H), lambda i: (0, 0)),
            pl.BlockSpec((1, H), lambda i: (0, 0)),
        ],
        out_specs=[
            pl.BlockSpec((BN, H), lambda i: (i, 0)),
            pl.BlockSpec((BN, HH), lambda i: (i, 0)),
            pl.BlockSpec((BN, HH), lambda i: (i, 0)),
        ],
        out_shape=[
            jax.ShapeDtypeStruct((N, H), _F32),
            jax.ShapeDtypeStruct((N, HH), _F32),
            jax.ShapeDtypeStruct((N, HH), _F32),
        ],
    )(sp3, emb_p, w1, b1, w2, b2)


def _update(h, aa, ab, uw1_ref, ub1_ref, uw2_ref, ub2_ref):
    u = (jnp.dot(h, uw1_ref[0:H], preferred_element_type=_F32)
         + jnp.dot(aa, uw1_ref[H:H + HH], preferred_element_type=_F32)
         + jnp.dot(ab, uw1_ref[H + HH:2 * H], preferred_element_type=_F32)
         + ub1_ref[...])
    return h + jnp.dot(jax.nn.silu(u), uw2_ref[...],
                       preferred_element_type=_F32) + ub2_ref[...]


def _up_body(h_ref, aa_ref, ab_ref, uw1_ref, ub1_ref, uw2_ref, ub2_ref,
             mw1_ref, mb1_ref, mw2_ref, mb2_ref, hn_ref, ma_ref, mb_ref):
    hn = _update(h_ref[...], aa_ref[...], ab_ref[...],
                 uw1_ref, ub1_ref, uw2_ref, ub2_ref)
    hn_ref[...] = hn
    m = _mlp(hn, mw1_ref[...], mb1_ref[...], mw2_ref[...], mb2_ref[...])
    ma_ref[...] = m[:, :HH]
    mb_ref[...] = m[:, HH:]


def _up_call(h, aa, ab, uw1, ub1, uw2, ub2, mw1, mb1, mw2, mb2):
    return pl.pallas_call(
        _up_body,
        grid=(N // BN,),
        in_specs=[
            pl.BlockSpec((BN, H), lambda i: (i, 0)),
            pl.BlockSpec((BN, HH), lambda i: (i, 0)),
            pl.BlockSpec((BN, HH), lambda i: (i, 0)),
            pl.BlockSpec((2 * H, H), lambda i: (0, 0)),
            pl.BlockSpec((1, H), lambda i: (0, 0)),
            pl.BlockSpec((H, H), lambda i: (0, 0)),
            pl.BlockSpec((1, H), lambda i: (0, 0)),
            pl.BlockSpec((H, H), lambda i: (0, 0)),
            pl.BlockSpec((1, H), lambda i: (0, 0)),
            pl.BlockSpec((H, H), lambda i: (0, 0)),
            pl.BlockSpec((1, H), lambda i: (0, 0)),
        ],
        out_specs=[
            pl.BlockSpec((BN, H), lambda i: (i, 0)),
            pl.BlockSpec((BN, HH), lambda i: (i, 0)),
            pl.BlockSpec((BN, HH), lambda i: (i, 0)),
        ],
        out_shape=[
            jax.ShapeDtypeStruct((N, H), _F32),
            jax.ShapeDtypeStruct((N, HH), _F32),
            jax.ShapeDtypeStruct((N, HH), _F32),
        ],
    )(h, aa, ab, uw1, ub1, uw2, ub2, mw1, mb1, mw2, mb2)


def _k3_body(h_ref, aa_ref, ab_ref, uw1_ref, ub1_ref, uw2_ref, ub2_ref,
             ow1_ref, ob1_ref, ow2_ref, ob2_ref, st_ref, preds_ref):
    pi = pl.program_id(0)
    hn = _update(h_ref[...], aa_ref[...], ab_ref[...],
                 uw1_ref, ub1_ref, uw2_ref, ub2_ref)
    t = jax.nn.silu(jnp.dot(hn, ow1_ref[...], preferred_element_type=_F32)
                    + ob1_ref[...])
    ao = jnp.dot(t, ow2_ref[...], preferred_element_type=_F32) + ob2_ref[...]
    ii = lax.broadcasted_iota(jnp.int32, (144, BN), 0).astype(_F32)
    pp = (lax.broadcasted_iota(jnp.int32, (144, BN), 1)
          + pi * BN).astype(_F32)
    s = st_ref[...]
    mask = ((pp >= s) & (pp < s + ii)).astype(_F32)
    part = jnp.dot(mask, ao, preferred_element_type=_F32)

    @pl.when(pi == 0)
    def _():
        preds_ref[...] = jnp.zeros_like(preds_ref)

    preds_ref[...] += part


def _k3_call(h, aa, ab, uw1, ub1, uw2, ub2, ow1, ob1, ow2_p, ob2_p, st_p):
    return pl.pallas_call(
        _k3_body,
        grid=(N // BN,),
        in_specs=[
            pl.BlockSpec((BN, H), lambda i: (i, 0)),
            pl.BlockSpec((BN, HH), lambda i: (i, 0)),
            pl.BlockSpec((BN, HH), lambda i: (i, 0)),
            pl.BlockSpec((2 * H, H), lambda i: (0, 0)),
            pl.BlockSpec((1, H), lambda i: (0, 0)),
            pl.BlockSpec((H, H), lambda i: (0, 0)),
            pl.BlockSpec((1, H), lambda i: (0, 0)),
            pl.BlockSpec((H, H), lambda i: (0, 0)),
            pl.BlockSpec((1, H), lambda i: (0, 0)),
            pl.BlockSpec((H, 8), lambda i: (0, 0)),
            pl.BlockSpec((1, 8), lambda i: (0, 0)),
            pl.BlockSpec((144, 1), lambda i: (0, 0)),
        ],
        out_specs=pl.BlockSpec((144, 8), lambda i: (0, 0)),
        out_shape=jax.ShapeDtypeStruct((144, 8), _F32),
    )(h, aa, ab, uw1, ub1, uw2, ub2, ow1, ob1, ow2_p, ob2_p, st_p)


def _edge_body(ma_hbm, mbh_hbm, da_hbm, dbh_hbm, src_hbm, tgt_hbm,
               oa_hbm, ob_hbm,
               sidx0, sidx1, tidx0, tidx1, grow0, grow1, drow0, drow1,
               zbuf, agg_sh, isem0, isem1, gsem0, gsem1, dsem0, dsem1):
    c = lax.axis_index("c")
    s = lax.axis_index("s")
    sidx = (sidx0, sidx1)
    tidx = (tidx0, tidx1)
    grow = (grow0, grow1)
    drow = (drow0, drow1)
    isem = (isem0, isem1)
    gsem = (gsem0, gsem1)
    dsem = (dsem0, dsem1)
    base0 = s * EPT

    def _zrow(r, carry):
        for cc in range(HH // 16):
            zbuf[r, pl.ds(cc * 16, 16)] = jnp.zeros((16,), _F32)
        return carry

    lax.fori_loop(0, ZR, _zrow, 0)

    def _zcp(j, carry):
        ch = j * NS + s

        @pl.when(ch < NZCH)
        def _():
            pltpu.sync_copy(zbuf, agg_sh.at[pl.ds(ch * ZR, ZR)])

        return carry

    lax.fori_loop(0, ZPT, _zcp, 0)
    plsc.subcore_barrier()

    def _idx_start(k, b):
        pltpu.async_copy(src_hbm.at[pl.ds(base0 + k * CH, CH)], sidx[b],
                         isem[b])
        pltpu.async_copy(tgt_hbm.at[pl.ds(base0 + k * CH, CH)], tidx[b],
                         isem[b])

    def _idx_wait(k, b):
        pltpu.make_async_copy(src_hbm.at[pl.ds(base0 + k * CH, CH)], sidx[b],
                              isem[b]).wait()
        pltpu.make_async_copy(tgt_hbm.at[pl.ds(base0 + k * CH, CH)], tidx[b],
                              isem[b]).wait()

    def _half(m_hbm, d_hbm, o_hbm):
        base0h = s * (EPT // 2)
        CHH = CH // 2

        def _load_start(k, b):
            pltpu.async_copy(m_hbm.at[sidx[b]], grow[b], gsem[b])
            pltpu.async_copy(d_hbm.at[pl.ds(base0h + k * CHH, CHH)], drow[b],
                             dsem[b])

        def _load_wait(k, b):
            pltpu.make_async_copy(m_hbm.at[sidx[b]], grow[b], gsem[b]).wait()
            pltpu.make_async_copy(d_hbm.at[pl.ds(base0h + k * CHH, CHH)],
                                  drow[b], dsem[b]).wait()

        def _section(k, b):
            @pl.when(k + 1 < NCHUNK)
            def _():
                _idx_wait(k + 1, 1 - b)
                _load_start(k + 1, 1 - b)

            _load_wait(k, b)

            shv = jnp.full((16,), 16, jnp.int32)
            mkv = jnp.full((16,), -65536, jnp.int32)

            def _mul(r2, cr):
                for cc in range(HH // 16):
                    sl = pl.ds(cc * 16, 16)
                    di = drow[b][r2, sl]
                    dlo = lax.bitcast_convert_type(
                        lax.shift_left(di, shv), _F32)
                    dhi = lax.bitcast_convert_type(
                        lax.bitwise_and(di, mkv), _F32)
                    grow[b][2 * r2, sl] = grow[b][2 * r2, sl] * dlo
                    grow[b][2 * r2 + 1, sl] = grow[b][2 * r2 + 1, sl] * dhi
                return cr

            lax.fori_loop(0, CHH, _mul, 0)
            pltpu.sync_copy(grow[b], agg_sh.at[tidx[b]], add=True)

            @pl.when(k + 2 < NCHUNK)
            def _():
                _idx_start(k + 2, b)

        _idx_start(0, 0)
        _idx_start(1, 1)
        _idx_wait(0, 0)
        _load_start(0, 0)

        def _body(j, carry):
            for b in (0, 1):
                _section(j * 2 + b, b)
            return carry

        lax.fori_loop(0, NCHUNK // 2, _body, 0)
        _section(jnp.int32(NCHUNK - 1), (NCHUNK - 1) % 2)

        plsc.subcore_barrier()

        def _wb(j, carry):
            ch = j * NS + s

            @pl.when(ch < NZCH)
            def _():
                pltpu.sync_copy(agg_sh.at[pl.ds(ch * ZR, ZR)], zbuf)
                pltpu.sync_copy(zbuf, o_hbm.at[pl.ds(ch * ZR, ZR)])

            return carry

        lax.fori_loop(0, ZPT, _wb, 0)

    @pl.when(c == 0)
    def _():
        _half(ma_hbm, da_hbm, oa_hbm)

    @pl.when(c == 1)
    def _():
        _half(mbh_hbm, dbh_hbm, ob_hbm)


def _edge_call(ma, mbh, dfa, dfb, src, tgt):
    kern = pl.kernel(
        _edge_body,
        out_type=[jax.ShapeDtypeStruct((N, HH), _F32)] * 2,
        mesh=plsc.VectorSubcoreMesh(core_axis_name="c", subcore_axis_name="s"),
        scratch_types=[
            pltpu.VMEM((CH,), jnp.int32),
            pltpu.VMEM((CH,), jnp.int32),
            pltpu.VMEM((CH,), jnp.int32),
            pltpu.VMEM((CH,), jnp.int32),
            pltpu.VMEM((CH, HH), _F32),
            pltpu.VMEM((CH, HH), _F32),
            pltpu.VMEM((CH // 2, HH), jnp.int32),
            pltpu.VMEM((CH // 2, HH), jnp.int32),
            pltpu.VMEM((ZR, HH), _F32),
            pltpu.VMEM_SHARED((N, HH), _F32),
            pltpu.SemaphoreType.DMA,
            pltpu.SemaphoreType.DMA,
            pltpu.SemaphoreType.DMA,
            pltpu.SemaphoreType.DMA,
            pltpu.SemaphoreType.DMA,
            pltpu.SemaphoreType.DMA,
        ],
    )
    return kern(ma, mbh, dfa, dfb, src, tgt)


def kernel(species, pcmbdf_feats, edge_index, edge_dist, n_atoms_list, embed,
           dW1, db1, dW2, db2, mW1, mb1, mW2, mb2,
           uW1, ub1, uW2, ub2, oW1, ob1, oW2, ob2):
    species = species.astype(jnp.int32)
    src = edge_index[0].astype(jnp.int32)
    tgt = edge_index[1].astype(jnp.int32)
    d2 = edge_dist[:, None].astype(_F32)
    emb_p = jnp.zeros((128, H), _F32).at[:NSP, :].set(embed.astype(_F32))

    na = n_atoms_list.astype(_F32)
    starts = jnp.concatenate([jnp.zeros((1,), _F32), jnp.cumsum(na)[:-1]])
    st_p = jnp.concatenate([starts, jnp.full((3,), 2.0 * N, _F32)])[:, None]

    ow2_p = jnp.zeros((H, 8), _F32).at[:, 0].set(oW2[:, 0])
    ob2_p = jnp.zeros((1, 8), _F32).at[0, 0].set(ob2[0])

    dfa, dfb = _dist_call(d2, dW1, db1[None], dW2, db2[None])
    h, ma, mbh = _k0_call(species.reshape(N // BN, 1, BN), emb_p,
                          mW1[0], mb1[0][None], mW2[0], mb2[0][None])
    preds = None
    for l in range(L):
        aa, ab = _edge_call(ma, mbh, dfa, dfb, src, tgt)
        if l + 1 < L:
            h, ma, mbh = _up_call(h, aa, ab,
                                  uW1[l], ub1[l][None], uW2[l], ub2[l][None],
                                  mW1[l + 1], mb1[l + 1][None],
                                  mW2[l + 1], mb2[l + 1][None])
        else:
            preds = _k3_call(h, aa, ab,
                             uW1[l], ub1[l][None], uW2[l], ub2[l][None],
                             oW1, ob1[None], ow2_p, ob2_p, st_p)
    return preds[:141, 0]


# speedup vs baseline: 1.5928x; 1.5928x over previous
import functools

import jax
import jax.numpy as jnp
from jax import lax
from jax.experimental import pallas as pl
from jax.experimental.pallas import tpu as pltpu
from jax.experimental.pallas import tpu_sc as plsc

N = 10000
E = 160000
H = 256
HH = 128
L = 3
NSP = 100
CUTOFF = 6.0

BN = 2000
BE = 2000

NS = 16
EPT = E // NS
CH = 80
NCHUNK = EPT // CH
MUR = 4
ZR = 40
NZCH = N // ZR
ZPT = -(-NZCH // NS)
_F32 = jnp.float32


def _mlp(x, w1, b1, w2, b2):
    t = jax.nn.silu(jnp.dot(x, w1, preferred_element_type=_F32) + b1)
    return jnp.dot(t, w2, preferred_element_type=_F32) + b2


def _dist_body(d_ref, w1_ref, b1_ref, w2_ref, b2_ref, oa_ref, ob_ref):
    d = d_ref[...]
    env = 0.5 * (jnp.cos((jnp.pi / CUTOFF) * d) + 1.0)
    t = jax.nn.silu(d * w1_ref[...] + b1_ref[...])
    x = jnp.dot(t, w2_ref[...], preferred_element_type=_F32) + b2_ref[...]
    x = x * env
    oa_ref[...] = x[:, :HH]
    ob_ref[...] = x[:, HH:]


def _dist_call(d2, w1, b1, w2, b2):
    return pl.pallas_call(
        _dist_body,
        grid=(E // BE,),
        in_specs=[
            pl.BlockSpec((BE, 1), lambda i: (i, 0)),
            pl.BlockSpec((1, H), lambda i: (0, 0)),
            pl.BlockSpec((1, H), lambda i: (0, 0)),
            pl.BlockSpec((H, H), lambda i: (0, 0)),
            pl.BlockSpec((1, H), lambda i: (0, 0)),
        ],
        out_specs=[
            pl.BlockSpec((BE, HH), lambda i: (i, 0)),
            pl.BlockSpec((BE, HH), lambda i: (i, 0)),
        ],
        out_shape=[jax.ShapeDtypeStruct((E, HH), _F32)] * 2,
    )(d2, w1, b1, w2, b2)


def _k0_body(sp_ref, emb_ref, w1_ref, b1_ref, w2_ref, b2_ref,
             h_ref, ma_ref, mb_ref):
    sp = sp_ref[0, 0]
    oh = (sp[:, None] == lax.broadcasted_iota(jnp.int32, (BN, 128), 1))
    h0 = jnp.dot(oh.astype(_F32), emb_ref[...], preferred_element_type=_F32)
    h_ref[...] = h0
    m = _mlp(h0, w1_ref[...], b1_ref[...], w2_ref[...], b2_ref[...])
    ma_ref[...] = m[:, :HH]
    mb_ref[...] = m[:, HH:]


def _k0_call(sp3, emb_p, w1, b1, w2, b2):
    return pl.pallas_call(
        _k0_body,
        grid=(N // BN,),
        in_specs=[
            pl.BlockSpec((1, 1, BN), lambda i: (i, 0, 0)),
            pl.BlockSpec((128, H), lambda i: (0, 0)),
            pl.BlockSpec((H, H), lambda i: (0, 0)),
            pl.BlockSpec((1, H), lambda i: (0, 0)),
            pl.BlockSpec((H, H), lambda i: (0, 0)),
            pl.BlockSpec((1, H), lambda i: (0, 0)),
        ],
        out_specs=[
            pl.BlockSpec((BN, H), lambda i: (i, 0)),
            pl.BlockSpec((BN, HH), lambda i: (i, 0)),
            pl.BlockSpec((BN, HH), lambda i: (i, 0)),
        ],
        out_shape=[
            jax.ShapeDtypeStruct((N, H), _F32),
            jax.ShapeDtypeStruct((N, HH), _F32),
            jax.ShapeDtypeStruct((N, HH), _F32),
        ],
    )(sp3, emb_p, w1, b1, w2, b2)


def _update(h, aa, ab, uw1_ref, ub1_ref, uw2_ref, ub2_ref):
    u = (jnp.dot(h, uw1_ref[0:H], preferred_element_type=_F32)
         + jnp.dot(aa, uw1_ref[H:H + HH], preferred_element_type=_F32)
         + jnp.dot(ab, uw1_ref[H + HH:2 * H], preferred_element_type=_F32)
         + ub1_ref[...])
    return h + jnp.dot(jax.nn.silu(u), uw2_ref[...],
                       preferred_element_type=_F32) + ub2_ref[...]


def _up_body(h_ref, aa_ref, ab_ref, uw1_ref, ub1_ref, uw2_ref, ub2_ref,
             mw1_ref, mb1_ref, mw2_ref, mb2_ref, hn_ref, ma_ref, mb_ref):
    hn = _update(h_ref[...], aa_ref[...], ab_ref[...],
                 uw1_ref, ub1_ref, uw2_ref, ub2_ref)
    hn_ref[...] = hn
    m = _mlp(hn, mw1_ref[...], mb1_ref[...], mw2_ref[...], mb2_ref[...])
    ma_ref[...] = m[:, :HH]
    mb_ref[...] = m[:, HH:]


def _up_call(h, aa, ab, uw1, ub1, uw2, ub2, mw1, mb1, mw2, mb2):
    return pl.pallas_call(
        _up_body,
        grid=(N // BN,),
        in_specs=[
            pl.BlockSpec((BN, H), lambda i: (i, 0)),
            pl.BlockSpec((BN, HH), lambda i: (i, 0)),
            pl.BlockSpec((BN, HH), lambda i: (i, 0)),
            pl.BlockSpec((2 * H, H), lambda i: (0, 0)),
            pl.BlockSpec((1, H), lambda i: (0, 0)),
            pl.BlockSpec((H, H), lambda i: (0, 0)),
            pl.BlockSpec((1, H), lambda i: (0, 0)),
            pl.BlockSpec((H, H), lambda i: (0, 0)),
            pl.BlockSpec((1, H), lambda i: (0, 0)),
            pl.BlockSpec((H, H), lambda i: (0, 0)),
            pl.BlockSpec((1, H), lambda i: (0, 0)),
        ],
        out_specs=[
            pl.BlockSpec((BN, H), lambda i: (i, 0)),
            pl.BlockSpec((BN, HH), lambda i: (i, 0)),
            pl.BlockSpec((BN, HH), lambda i: (i, 0)),
        ],
        out_shape=[
            jax.ShapeDtypeStruct((N, H), _F32),
            jax.ShapeDtypeStruct((N, HH), _F32),
            jax.ShapeDtypeStruct((N, HH), _F32),
        ],
    )(h, aa, ab, uw1, ub1, uw2, ub2, mw1, mb1, mw2, mb2)


def _k3_body(h_ref, aa_ref, ab_ref, uw1_ref, ub1_ref, uw2_ref, ub2_ref,
             ow1_ref, ob1_ref, ow2_ref, ob2_ref, st_ref, preds_ref):
    pi = pl.program_id(0)
    hn = _update(h_ref[...], aa_ref[...], ab_ref[...],
                 uw1_ref, ub1_ref, uw2_ref, ub2_ref)
    t = jax.nn.silu(jnp.dot(hn, ow1_ref[...], preferred_element_type=_F32)
                    + ob1_ref[...])
    ao = jnp.dot(t, ow2_ref[...], preferred_element_type=_F32) + ob2_ref[...]
    ii = lax.broadcasted_iota(jnp.int32, (144, BN), 0).astype(_F32)
    pp = (lax.broadcasted_iota(jnp.int32, (144, BN), 1)
          + pi * BN).astype(_F32)
    s = st_ref[...]
    mask = ((pp >= s) & (pp < s + ii)).astype(_F32)
    part = jnp.dot(mask, ao, preferred_element_type=_F32)

    @pl.when(pi == 0)
    def _():
        preds_ref[...] = jnp.zeros_like(preds_ref)

    preds_ref[...] += part


def _k3_call(h, aa, ab, uw1, ub1, uw2, ub2, ow1, ob1, ow2_p, ob2_p, st_p):
    return pl.pallas_call(
        _k3_body,
        grid=(N // BN,),
        in_specs=[
            pl.BlockSpec((BN, H), lambda i: (i, 0)),
            pl.BlockSpec((BN, HH), lambda i: (i, 0)),
            pl.BlockSpec((BN, HH), lambda i: (i, 0)),
            pl.BlockSpec((2 * H, H), lambda i: (0, 0)),
            pl.BlockSpec((1, H), lambda i: (0, 0)),
            pl.BlockSpec((H, H), lambda i: (0, 0)),
            pl.BlockSpec((1, H), lambda i: (0, 0)),
            pl.BlockSpec((H, H), lambda i: (0, 0)),
            pl.BlockSpec((1, H), lambda i: (0, 0)),
            pl.BlockSpec((H, 8), lambda i: (0, 0)),
            pl.BlockSpec((1, 8), lambda i: (0, 0)),
            pl.BlockSpec((144, 1), lambda i: (0, 0)),
        ],
        out_specs=pl.BlockSpec((144, 8), lambda i: (0, 0)),
        out_shape=jax.ShapeDtypeStruct((144, 8), _F32),
    )(h, aa, ab, uw1, ub1, uw2, ub2, ow1, ob1, ow2_p, ob2_p, st_p)


def _edge_body(ma_hbm, mbh_hbm, da_hbm, dbh_hbm, src_hbm, tgt_hbm,
               oa_hbm, ob_hbm,
               sidx0, sidx1, tidx0, tidx1, grow0, grow1, drow0, drow1,
               zbuf, agg_sh, isem0, isem1, gsem0, gsem1, dsem0, dsem1):
    c = lax.axis_index("c")
    s = lax.axis_index("s")
    sidx = (sidx0, sidx1)
    tidx = (tidx0, tidx1)
    grow = (grow0, grow1)
    drow = (drow0, drow1)
    isem = (isem0, isem1)
    gsem = (gsem0, gsem1)
    dsem = (dsem0, dsem1)
    base0 = s * EPT

    def _zrow(r, carry):
        for cc in range(HH // 16):
            zbuf[r, pl.ds(cc * 16, 16)] = jnp.zeros((16,), _F32)
        return carry

    lax.fori_loop(0, ZR, _zrow, 0)

    def _zcp(j, carry):
        ch = j * NS + s

        @pl.when(ch < NZCH)
        def _():
            pltpu.sync_copy(zbuf, agg_sh.at[pl.ds(ch * ZR, ZR)])

        return carry

    lax.fori_loop(0, ZPT, _zcp, 0)
    plsc.subcore_barrier()

    def _idx_start(k, b):
        pltpu.async_copy(src_hbm.at[pl.ds(base0 + k * CH, CH)], sidx[b],
                         isem[b])
        pltpu.async_copy(tgt_hbm.at[pl.ds(base0 + k * CH, CH)], tidx[b],
                         isem[b])

    def _idx_wait(k, b):
        pltpu.make_async_copy(src_hbm.at[pl.ds(base0 + k * CH, CH)], sidx[b],
                              isem[b]).wait()
        pltpu.make_async_copy(tgt_hbm.at[pl.ds(base0 + k * CH, CH)], tidx[b],
                              isem[b]).wait()

    def _half(m_hbm, d_hbm, o_hbm):
        def _load_start(k, b):
            pltpu.async_copy(m_hbm.at[sidx[b]], grow[b], gsem[b])
            pltpu.async_copy(d_hbm.at[pl.ds(base0 + k * CH, CH)], drow[b],
                             dsem[b])

        def _load_wait(k, b):
            pltpu.make_async_copy(m_hbm.at[sidx[b]], grow[b], gsem[b]).wait()
            pltpu.make_async_copy(d_hbm.at[pl.ds(base0 + k * CH, CH)],
                                  drow[b], dsem[b]).wait()

        def _section(k, b):
            @pl.when(k + 1 < NCHUNK)
            def _():
                _idx_wait(k + 1, 1 - b)
                _load_start(k + 1, 1 - b)

            _load_wait(k, b)

            def _mul(r4, cr):
                for rr in range(MUR):
                    r = r4 * MUR + rr
                    for cc in range(HH // 16):
                        sl = pl.ds(cc * 16, 16)
                        grow[b][r, sl] = grow[b][r, sl] * drow[b][r, sl]
                return cr

            lax.fori_loop(0, CH // MUR, _mul, 0)
            pltpu.sync_copy(grow[b], agg_sh.at[tidx[b]], add=True)

            @pl.when(k + 2 < NCHUNK)
            def _():
                _idx_start(k + 2, b)

        _idx_start(0, 0)
        _idx_start(1, 1)
        _idx_wait(0, 0)
        _load_start(0, 0)

        def _body(j, carry):
            for b in (0, 1):
                _section(j * 2 + b, b)
            return carry

        lax.fori_loop(0, NCHUNK // 2, _body, 0)
        if NCHUNK % 2 == 1:
            _section(jnp.int32(NCHUNK - 1), (NCHUNK - 1) % 2)

        plsc.subcore_barrier()

        def _wb(j, carry):
            ch = j * NS + s

            @pl.when(ch < NZCH)
            def _():
                pltpu.sync_copy(agg_sh.at[pl.ds(ch * ZR, ZR)], zbuf)
                pltpu.sync_copy(zbuf, o_hbm.at[pl.ds(ch * ZR, ZR)])

            return carry

        lax.fori_loop(0, ZPT, _wb, 0)

    @pl.when(c == 0)
    def _():
        _half(ma_hbm, da_hbm, oa_hbm)

    @pl.when(c == 1)
    def _():
        _half(mbh_hbm, dbh_hbm, ob_hbm)


def _edge_call(ma, mbh, dfa, dfb, src, tgt):
    kern = pl.kernel(
        _edge_body,
        out_type=[jax.ShapeDtypeStruct((N, HH), _F32)] * 2,
        mesh=plsc.VectorSubcoreMesh(core_axis_name="c", subcore_axis_name="s"),
        scratch_types=[
            pltpu.VMEM((CH,), jnp.int32),
            pltpu.VMEM((CH,), jnp.int32),
            pltpu.VMEM((CH,), jnp.int32),
            pltpu.VMEM((CH,), jnp.int32),
            pltpu.VMEM((CH, HH), _F32),
            pltpu.VMEM((CH, HH), _F32),
            pltpu.VMEM((CH, HH), _F32),
            pltpu.VMEM((CH, HH), _F32),
            pltpu.VMEM((ZR, HH), _F32),
            pltpu.VMEM_SHARED((N, HH), _F32),
            pltpu.SemaphoreType.DMA,
            pltpu.SemaphoreType.DMA,
            pltpu.SemaphoreType.DMA,
            pltpu.SemaphoreType.DMA,
            pltpu.SemaphoreType.DMA,
            pltpu.SemaphoreType.DMA,
        ],
    )
    return kern(ma, mbh, dfa, dfb, src, tgt)


def kernel(species, pcmbdf_feats, edge_index, edge_dist, n_atoms_list, embed,
           dW1, db1, dW2, db2, mW1, mb1, mW2, mb2,
           uW1, ub1, uW2, ub2, oW1, ob1, oW2, ob2):
    species = species.astype(jnp.int32)
    src = edge_index[0].astype(jnp.int32)
    tgt = edge_index[1].astype(jnp.int32)
    d2 = edge_dist[:, None].astype(_F32)
    emb_p = jnp.zeros((128, H), _F32).at[:NSP, :].set(embed.astype(_F32))

    na = n_atoms_list.astype(_F32)
    starts = jnp.concatenate([jnp.zeros((1,), _F32), jnp.cumsum(na)[:-1]])
    st_p = jnp.concatenate([starts, jnp.full((3,), 2.0 * N, _F32)])[:, None]

    ow2_p = jnp.zeros((H, 8), _F32).at[:, 0].set(oW2[:, 0])
    ob2_p = jnp.zeros((1, 8), _F32).at[0, 0].set(ob2[0])

    dfa, dfb = _dist_call(d2, dW1, db1[None], dW2, db2[None])
    h, ma, mbh = _k0_call(species.reshape(N // BN, 1, BN), emb_p,
                          mW1[0], mb1[0][None], mW2[0], mb2[0][None])
    preds = None
    for l in range(L):
        aa, ab = _edge_call(ma, mbh, dfa, dfb, src, tgt)
        if l + 1 < L:
            h, ma, mbh = _up_call(h, aa, ab,
                                  uW1[l], ub1[l][None], uW2[l], ub2[l][None],
                                  mW1[l + 1], mb1[l + 1][None],
                                  mW2[l + 1], mb2[l + 1][None])
        else:
            preds = _k3_call(h, aa, ab,
                             uW1[l], ub1[l][None], uW2[l], ub2[l][None],
                             oW1, ob1[None], ow2_p, ob2_p, st_p)
    return preds[:141, 0]


# speedup vs baseline: 1.6244x; 1.0198x over previous
import functools

import jax
import jax.numpy as jnp
from jax import lax
from jax.experimental import pallas as pl
from jax.experimental.pallas import tpu as pltpu
from jax.experimental.pallas import tpu_sc as plsc

N = 10000
E = 160000
H = 256
HH = 128
L = 3
NSP = 100
CUTOFF = 6.0

BN = 2000
BE = 2000

NS = 16
EPT = E // NS
CH = 80
NCHUNK = EPT // CH
MUR = 4
ZR = 40
NZCH = N // ZR
ZPT = -(-NZCH // NS)
_F32 = jnp.float32


def _silu(x):
    return 0.5 * x * (jnp.tanh(0.5 * x) + 1.0)


def _mlp(x, w1, b1, w2, b2):
    t = _silu(jnp.dot(x, w1, preferred_element_type=_F32) + b1)
    return jnp.dot(t, w2, preferred_element_type=_F32) + b2


def _pack_half(xh):
    lo = lax.bitcast_convert_type(xh[:, :64].astype(jnp.bfloat16),
                                  jnp.uint16).astype(jnp.uint32)
    hi = lax.bitcast_convert_type(xh[:, 64:].astype(jnp.bfloat16),
                                  jnp.uint16).astype(jnp.uint32)
    return lax.bitcast_convert_type(lo | (hi << 16), jnp.int32)


def _dist_body(d_ref, w1_ref, b1_ref, w2_ref, b2_ref, oa_ref, ob_ref):
    d = d_ref[...]
    env = 0.5 * (jnp.cos((jnp.pi / CUTOFF) * d) + 1.0)
    t = _silu(d * w1_ref[...] + b1_ref[...])
    x = jnp.dot(t, w2_ref[...], preferred_element_type=_F32) + b2_ref[...]
    x = x * env
    oa_ref[...] = _pack_half(x[:, :HH])
    ob_ref[...] = _pack_half(x[:, HH:])


def _dist_call(d2, w1, b1, w2, b2):
    return pl.pallas_call(
        _dist_body,
        grid=(E // BE,),
        in_specs=[
            pl.BlockSpec((BE, 1), lambda i: (i, 0)),
            pl.BlockSpec((1, H), lambda i: (0, 0)),
            pl.BlockSpec((1, H), lambda i: (0, 0)),
            pl.BlockSpec((H, H), lambda i: (0, 0)),
            pl.BlockSpec((1, H), lambda i: (0, 0)),
        ],
        out_specs=[
            pl.BlockSpec((BE, HH // 2), lambda i: (i, 0)),
            pl.BlockSpec((BE, HH // 2), lambda i: (i, 0)),
        ],
        out_shape=[jax.ShapeDtypeStruct((E, HH // 2), jnp.int32)] * 2,
    )(d2, w1, b1, w2, b2)


def _k0_body(sp_ref, emb_ref, w1_ref, b1_ref, w2_ref, b2_ref,
             h_ref, ma_ref, mb_ref):
    sp = sp_ref[0, 0]
    oh = (sp[:, None] == lax.broadcasted_iota(jnp.int32, (BN, 128), 1))
    h0 = jnp.dot(oh.astype(_F32), emb_ref[...], preferred_element_type=_F32)
    h_ref[...] = h0
    m = _mlp(h0, w1_ref[...], b1_ref[...], w2_ref[...], b2_ref[...])
    ma_ref[...] = m[:, :HH]
    mb_ref[...] = m[:, HH:]


def _k0_call(sp3, emb_p, w1, b1, w2, b2):
    return pl.pallas_call(
        _k0_body,
        grid=(N // BN,),
        in_specs=[
            pl.BlockSpec((1, 1, BN), lambda i: (i, 0, 0)),
            pl.BlockSpec((128, H), lambda i: (0, 0)),
            pl.BlockSpec((H, H), lambda i: (0, 0)),
            pl.BlockSpec((1, H), lambda i: (0, 0)),
            pl.BlockSpec((H, H), lambda i: (0, 0)),
            pl.BlockSpec((1, H), lambda i: (0, 0)),
        ],
        out_specs=[
            pl.BlockSpec((BN, H), lambda i: (i, 0)),
            pl.BlockSpec((BN, HH), lambda i: (i, 0)),
            pl.BlockSpec((BN, HH), lambda i: (i, 0)),
        ],
        out_shape=[
            jax.ShapeDtypeStruct((N, H), _F32),
            jax.ShapeDtypeStruct((N, HH), _F32),
            jax.ShapeDtypeStruct((N, HH), _F32),
        ],
    )(sp3, emb_p, w1, b1, w2, b2)


def _update(h, aa, ab, uw1_ref, ub1_ref, uw2_ref, ub2_ref):
    u = (jnp.dot(h, uw1_ref[0:H], preferred_element_type=_F32)
         + jnp.dot(aa, uw1_ref[H:H + HH], preferred_element_type=_F32)
         + jnp.dot(ab, uw1_ref[H + HH:2 * H], preferred_element_type=_F32)
         + ub1_ref[...])
    return h + jnp.dot(_silu(u), uw2_ref[...],
                       preferred_element_type=_F32) + ub2_ref[...]


def _up_body(h_ref, aa_ref, ab_ref, uw1_ref, ub1_ref, uw2_ref, ub2_ref,
             mw1_ref, mb1_ref, mw2_ref, mb2_ref, hn_ref, ma_ref, mb_ref):
    hn = _update(h_ref[...], aa_ref[...], ab_ref[...],
                 uw1_ref, ub1_ref, uw2_ref, ub2_ref)
    hn_ref[...] = hn
    m = _mlp(hn, mw1_ref[...], mb1_ref[...], mw2_ref[...], mb2_ref[...])
    ma_ref[...] = m[:, :HH]
    mb_ref[...] = m[:, HH:]


def _up_call(h, aa, ab, uw1, ub1, uw2, ub2, mw1, mb1, mw2, mb2):
    return pl.pallas_call(
        _up_body,
        grid=(N // BN,),
        in_specs=[
            pl.BlockSpec((BN, H), lambda i: (i, 0)),
            pl.BlockSpec((BN, HH), lambda i: (i, 0)),
            pl.BlockSpec((BN, HH), lambda i: (i, 0)),
            pl.BlockSpec((2 * H, H), lambda i: (0, 0)),
            pl.BlockSpec((1, H), lambda i: (0, 0)),
            pl.BlockSpec((H, H), lambda i: (0, 0)),
            pl.BlockSpec((1, H), lambda i: (0, 0)),
            pl.BlockSpec((H, H), lambda i: (0, 0)),
            pl.BlockSpec((1, H), lambda i: (0, 0)),
            pl.BlockSpec((H, H), lambda i: (0, 0)),
            pl.BlockSpec((1, H), lambda i: (0, 0)),
        ],
        out_specs=[
            pl.BlockSpec((BN, H), lambda i: (i, 0)),
            pl.BlockSpec((BN, HH), lambda i: (i, 0)),
            pl.BlockSpec((BN, HH), lambda i: (i, 0)),
        ],
        out_shape=[
            jax.ShapeDtypeStruct((N, H), _F32),
            jax.ShapeDtypeStruct((N, HH), _F32),
            jax.ShapeDtypeStruct((N, HH), _F32),
        ],
    )(h, aa, ab, uw1, ub1, uw2, ub2, mw1, mb1, mw2, mb2)


def _k3_body(h_ref, aa_ref, ab_ref, uw1_ref, ub1_ref, uw2_ref, ub2_ref,
             ow1_ref, ob1_ref, ow2_ref, ob2_ref, st_ref, preds_ref):
    pi = pl.program_id(0)
    hn = _update(h_ref[...], aa_ref[...], ab_ref[...],
                 uw1_ref, ub1_ref, uw2_ref, ub2_ref)
    t = _silu(jnp.dot(hn, ow1_ref[...], preferred_element_type=_F32)
              + ob1_ref[...])
    ao = jnp.dot(t, ow2_ref[...], preferred_element_type=_F32) + ob2_ref[...]
    ii = lax.broadcasted_iota(jnp.int32, (144, BN), 0).astype(_F32)
    pp = (lax.broadcasted_iota(jnp.int32, (144, BN), 1)
          + pi * BN).astype(_F32)
    s = st_ref[...]
    mask = ((pp >= s) & (pp < s + ii)).astype(_F32)
    part = jnp.dot(mask, ao, preferred_element_type=_F32)

    @pl.when(pi == 0)
    def _():
        preds_ref[...] = jnp.zeros_like(preds_ref)

    preds_ref[...] += part


def _k3_call(h, aa, ab, uw1, ub1, uw2, ub2, ow1, ob1, ow2_p, ob2_p, st_p):
    return pl.pallas_call(
        _k3_body,
        grid=(N // BN,),
        in_specs=[
            pl.BlockSpec((BN, H), lambda i: (i, 0)),
            pl.BlockSpec((BN, HH), lambda i: (i, 0)),
            pl.BlockSpec((BN, HH), lambda i: (i, 0)),
            pl.BlockSpec((2 * H, H), lambda i: (0, 0)),
            pl.BlockSpec((1, H), lambda i: (0, 0)),
            pl.BlockSpec((H, H), lambda i: (0, 0)),
            pl.BlockSpec((1, H), lambda i: (0, 0)),
            pl.BlockSpec((H, H), lambda i: (0, 0)),
            pl.BlockSpec((1, H), lambda i: (0, 0)),
            pl.BlockSpec((H, 8), lambda i: (0, 0)),
            pl.BlockSpec((1, 8), lambda i: (0, 0)),
            pl.BlockSpec((144, 1), lambda i: (0, 0)),
        ],
        out_specs=pl.BlockSpec((144, 8), lambda i: (0, 0)),
        out_shape=jax.ShapeDtypeStruct((144, 8), _F32),
    )(h, aa, ab, uw1, ub1, uw2, ub2, ow1, ob1, ow2_p, ob2_p, st_p)


def _edge_body(ma_hbm, mbh_hbm, da_hbm, dbh_hbm, src_hbm, tgt_hbm,
               oa_hbm, ob_hbm,
               sidx0, sidx1, tidx0, tidx1, grow0, grow1, drow0, drow1,
               zbuf, agg_sh, isem0, isem1, gsem0, gsem1, dsem0, dsem1):
    c = lax.axis_index("c")
    s = lax.axis_index("s")
    sidx = (sidx0, sidx1)
    tidx = (tidx0, tidx1)
    grow = (grow0, grow1)
    drow = (drow0, drow1)
    isem = (isem0, isem1)
    gsem = (gsem0, gsem1)
    dsem = (dsem0, dsem1)
    base0 = s * EPT

    def _zrow(r, carry):
        for cc in range(HH // 16):
            zbuf[r, pl.ds(cc * 16, 16)] = jnp.zeros((16,), _F32)
        return carry

    lax.fori_loop(0, ZR, _zrow, 0)

    def _zcp(j, carry):
        ch = j * NS + s

        @pl.when(ch < NZCH)
        def _():
            pltpu.sync_copy(zbuf, agg_sh.at[pl.ds(ch * ZR, ZR)])

        return carry

    lax.fori_loop(0, ZPT, _zcp, 0)
    plsc.subcore_barrier()

    def _idx_start(k, b):
        pltpu.async_copy(src_hbm.at[pl.ds(base0 + k * CH, CH)], sidx[b],
                         isem[b])
        pltpu.async_copy(tgt_hbm.at[pl.ds(base0 + k * CH, CH)], tidx[b],
                         isem[b])

    def _idx_wait(k, b):
        pltpu.make_async_copy(src_hbm.at[pl.ds(base0 + k * CH, CH)], sidx[b],
                              isem[b]).wait()
        pltpu.make_async_copy(tgt_hbm.at[pl.ds(base0 + k * CH, CH)], tidx[b],
                              isem[b]).wait()

    def _half(m_hbm, d_hbm, o_hbm):
        def _load_start(k, b):
            pltpu.async_copy(m_hbm.at[sidx[b]], grow[b], gsem[b])
            pltpu.async_copy(d_hbm.at[pl.ds(base0 + k * CH, CH)], drow[b],
                             dsem[b])

        def _load_wait(k, b):
            pltpu.make_async_copy(m_hbm.at[sidx[b]], grow[b], gsem[b]).wait()
            pltpu.make_async_copy(d_hbm.at[pl.ds(base0 + k * CH, CH)],
                                  drow[b], dsem[b]).wait()

        shv = jnp.full((16,), 16, jnp.int32)
        mkv = jnp.full((16,), -65536, jnp.int32)

        def _section(k, b):
            @pl.when(k + 1 < NCHUNK)
            def _():
                _idx_wait(k + 1, 1 - b)
                _load_start(k + 1, 1 - b)

            _load_wait(k, b)

            def _mul(r, cr):
                for cc in range(HH // 32):
                    di = drow[b][r, pl.ds(cc * 16, 16)]
                    dlo = lax.bitcast_convert_type(
                        lax.shift_left(di, shv), _F32)
                    dhi = lax.bitcast_convert_type(
                        lax.bitwise_and(di, mkv), _F32)
                    sl_lo = pl.ds(cc * 16, 16)
                    sl_hi = pl.ds(64 + cc * 16, 16)
                    grow[b][r, sl_lo] = grow[b][r, sl_lo] * dlo
                    grow[b][r, sl_hi] = grow[b][r, sl_hi] * dhi
                return cr

            lax.fori_loop(0, CH, _mul, 0)
            pltpu.sync_copy(grow[b], agg_sh.at[tidx[b]], add=True)

            @pl.when(k + 2 < NCHUNK)
            def _():
                _idx_start(k + 2, b)

        _idx_start(0, 0)
        _idx_start(1, 1)
        _idx_wait(0, 0)
        _load_start(0, 0)

        def _body(j, carry):
            for b in (0, 1):
                _section(j * 2 + b, b)
            return carry

        lax.fori_loop(0, NCHUNK // 2, _body, 0)
        if NCHUNK % 2 == 1:
            _section(jnp.int32(NCHUNK - 1), (NCHUNK - 1) % 2)

        plsc.subcore_barrier()

        def _wb(j, carry):
            ch = j * NS + s

            @pl.when(ch < NZCH)
            def _():
                pltpu.sync_copy(agg_sh.at[pl.ds(ch * ZR, ZR)], zbuf)
                pltpu.sync_copy(zbuf, o_hbm.at[pl.ds(ch * ZR, ZR)])

            return carry

        lax.fori_loop(0, ZPT, _wb, 0)

    @pl.when(c == 0)
    def _():
        _half(ma_hbm, da_hbm, oa_hbm)

    @pl.when(c == 1)
    def _():
        _half(mbh_hbm, dbh_hbm, ob_hbm)


def _edge_call(ma, mbh, dfa, dfb, src, tgt):
    kern = pl.kernel(
        _edge_body,
        out_type=[jax.ShapeDtypeStruct((N, HH), _F32)] * 2,
        mesh=plsc.VectorSubcoreMesh(core_axis_name="c", subcore_axis_name="s"),
        scratch_types=[
            pltpu.VMEM((CH,), jnp.int32),
            pltpu.VMEM((CH,), jnp.int32),
            pltpu.VMEM((CH,), jnp.int32),
            pltpu.VMEM((CH,), jnp.int32),
            pltpu.VMEM((CH, HH), _F32),
            pltpu.VMEM((CH, HH), _F32),
            pltpu.VMEM((CH, HH // 2), jnp.int32),
            pltpu.VMEM((CH, HH // 2), jnp.int32),
            pltpu.VMEM((ZR, HH), _F32),
            pltpu.VMEM_SHARED((N, HH), _F32),
            pltpu.SemaphoreType.DMA,
            pltpu.SemaphoreType.DMA,
            pltpu.SemaphoreType.DMA,
            pltpu.SemaphoreType.DMA,
            pltpu.SemaphoreType.DMA,
            pltpu.SemaphoreType.DMA,
        ],
    )
    return kern(ma, mbh, dfa, dfb, src, tgt)


def kernel(species, pcmbdf_feats, edge_index, edge_dist, n_atoms_list, embed,
           dW1, db1, dW2, db2, mW1, mb1, mW2, mb2,
           uW1, ub1, uW2, ub2, oW1, ob1, oW2, ob2):
    species = species.astype(jnp.int32)
    src = edge_index[0].astype(jnp.int32)
    tgt = edge_index[1].astype(jnp.int32)
    d2 = edge_dist[:, None].astype(_F32)
    emb_p = jnp.zeros((128, H), _F32).at[:NSP, :].set(embed.astype(_F32))

    na = n_atoms_list.astype(_F32)
    starts = jnp.concatenate([jnp.zeros((1,), _F32), jnp.cumsum(na)[:-1]])
    st_p = jnp.concatenate([starts, jnp.full((3,), 2.0 * N, _F32)])[:, None]

    ow2_p = jnp.zeros((H, 8), _F32).at[:, 0].set(oW2[:, 0])
    ob2_p = jnp.zeros((1, 8), _F32).at[0, 0].set(ob2[0])

    dfa, dfb = _dist_call(d2, dW1, db1[None], dW2, db2[None])
    h, ma, mbh = _k0_call(species.reshape(N // BN, 1, BN), emb_p,
                          mW1[0], mb1[0][None], mW2[0], mb2[0][None])
    preds = None
    for l in range(L):
        aa, ab = _edge_call(ma, mbh, dfa, dfb, src, tgt)
        if l + 1 < L:
            h, ma, mbh = _up_call(h, aa, ab,
                                  uW1[l], ub1[l][None], uW2[l], ub2[l][None],
                                  mW1[l + 1], mb1[l + 1][None],
                                  mW2[l + 1], mb2[l + 1][None])
        else:
            preds = _k3_call(h, aa, ab,
                             uW1[l], ub1[l][None], uW2[l], ub2[l][None],
                             oW1, ob1[None], ow2_p, ob2_p, st_p)
    return preds[:141, 0]


# speedup vs baseline: 1.7479x; 1.0761x over previous
import functools

import jax
import jax.numpy as jnp
from jax import lax
from jax.experimental import pallas as pl
from jax.experimental.pallas import tpu as pltpu
from jax.experimental.pallas import tpu_sc as plsc

N = 10000
E = 160000
H = 256
HH = 128
L = 3
NSP = 100
CUTOFF = 6.0

BN = 2000
BE = 2000

NS = 16
EPT = E // NS
CH = 80
NCHUNK = EPT // CH
MUR = 4
ZR = 40
NZCH = N // ZR
ZPT = -(-NZCH // NS)
_F32 = jnp.float32


def _silu(x):
    return 0.5 * x * (jnp.tanh(0.5 * x) + 1.0)


def _mlp(x, w1, b1, w2, b2):
    t = _silu(jnp.dot(x, w1, preferred_element_type=_F32) + b1)
    return jnp.dot(t, w2, preferred_element_type=_F32) + b2


def _pack_half(xh):
    lo = lax.bitcast_convert_type(xh[:, :64].astype(jnp.bfloat16),
                                  jnp.uint16).astype(jnp.uint32)
    hi = lax.bitcast_convert_type(xh[:, 64:].astype(jnp.bfloat16),
                                  jnp.uint16).astype(jnp.uint32)
    return lax.bitcast_convert_type(lo | (hi << 16), jnp.int32)


def _dist_body(d_ref, w1_ref, b1_ref, w2_ref, b2_ref, oa_ref, ob_ref):
    d = d_ref[...]
    env = 0.5 * (jnp.cos((jnp.pi / CUTOFF) * d) + 1.0)
    t = _silu(d * w1_ref[...] + b1_ref[...])
    x = jnp.dot(t, w2_ref[...], preferred_element_type=_F32) + b2_ref[...]
    x = x * env
    oa_ref[...] = _pack_half(x[:, :HH])
    ob_ref[...] = _pack_half(x[:, HH:])


def _dist_call(d2, w1, b1, w2, b2):
    return pl.pallas_call(
        _dist_body,
        grid=(E // BE,),
        in_specs=[
            pl.BlockSpec((BE, 1), lambda i: (i, 0)),
            pl.BlockSpec((1, H), lambda i: (0, 0)),
            pl.BlockSpec((1, H), lambda i: (0, 0)),
            pl.BlockSpec((H, H), lambda i: (0, 0)),
            pl.BlockSpec((1, H), lambda i: (0, 0)),
        ],
        out_specs=[
            pl.BlockSpec((BE, HH // 2), lambda i: (i, 0)),
            pl.BlockSpec((BE, HH // 2), lambda i: (i, 0)),
        ],
        out_shape=[jax.ShapeDtypeStruct((E, HH // 2), jnp.int32)] * 2,
    )(d2, w1, b1, w2, b2)


def _k0_body(sp_ref, emb_ref, w1_ref, b1_ref, w2_ref, b2_ref,
             h_ref, ma_ref, mb_ref):
    sp = sp_ref[0, 0]
    oh = (sp[:, None] == lax.broadcasted_iota(jnp.int32, (BN, 128), 1))
    h0 = jnp.dot(oh.astype(_F32), emb_ref[...], preferred_element_type=_F32)
    h_ref[...] = h0
    m = _mlp(h0, w1_ref[...], b1_ref[...], w2_ref[...], b2_ref[...])
    ma_ref[...] = m[:, :HH]
    mb_ref[...] = m[:, HH:]


def _k0_call(sp3, emb_p, w1, b1, w2, b2):
    return pl.pallas_call(
        _k0_body,
        grid=(N // BN,),
        in_specs=[
            pl.BlockSpec((1, 1, BN), lambda i: (i, 0, 0)),
            pl.BlockSpec((128, H), lambda i: (0, 0)),
            pl.BlockSpec((H, H), lambda i: (0, 0)),
            pl.BlockSpec((1, H), lambda i: (0, 0)),
            pl.BlockSpec((H, H), lambda i: (0, 0)),
            pl.BlockSpec((1, H), lambda i: (0, 0)),
        ],
        out_specs=[
            pl.BlockSpec((BN, H), lambda i: (i, 0)),
            pl.BlockSpec((BN, HH), lambda i: (i, 0)),
            pl.BlockSpec((BN, HH), lambda i: (i, 0)),
        ],
        out_shape=[
            jax.ShapeDtypeStruct((N, H), _F32),
            jax.ShapeDtypeStruct((N, HH), _F32),
            jax.ShapeDtypeStruct((N, HH), _F32),
        ],
    )(sp3, emb_p, w1, b1, w2, b2)


def _update(h, aa, ab, uw1_ref, ub1_ref, uw2_ref, ub2_ref):
    u = (jnp.dot(h, uw1_ref[0:H], preferred_element_type=_F32)
         + jnp.dot(aa, uw1_ref[H:H + HH], preferred_element_type=_F32)
         + jnp.dot(ab, uw1_ref[H + HH:2 * H], preferred_element_type=_F32)
         + ub1_ref[...])
    return h + jnp.dot(_silu(u), uw2_ref[...],
                       preferred_element_type=_F32) + ub2_ref[...]


def _up_body(h_ref, aa_ref, ab_ref, uw1_ref, ub1_ref, uw2_ref, ub2_ref,
             mw1_ref, mb1_ref, mw2_ref, mb2_ref, hn_ref, ma_ref, mb_ref):
    hn = _update(h_ref[...], aa_ref[...], ab_ref[...],
                 uw1_ref, ub1_ref, uw2_ref, ub2_ref)
    hn_ref[...] = hn
    m = _mlp(hn, mw1_ref[...], mb1_ref[...], mw2_ref[...], mb2_ref[...])
    ma_ref[...] = m[:, :HH]
    mb_ref[...] = m[:, HH:]


def _up_call(h, aa, ab, uw1, ub1, uw2, ub2, mw1, mb1, mw2, mb2):
    return pl.pallas_call(
        _up_body,
        grid=(N // BN,),
        in_specs=[
            pl.BlockSpec((BN, H), lambda i: (i, 0)),
            pl.BlockSpec((BN, HH), lambda i: (i, 0)),
            pl.BlockSpec((BN, HH), lambda i: (i, 0)),
            pl.BlockSpec((2 * H, H), lambda i: (0, 0)),
            pl.BlockSpec((1, H), lambda i: (0, 0)),
            pl.BlockSpec((H, H), lambda i: (0, 0)),
            pl.BlockSpec((1, H), lambda i: (0, 0)),
            pl.BlockSpec((H, H), lambda i: (0, 0)),
            pl.BlockSpec((1, H), lambda i: (0, 0)),
            pl.BlockSpec((H, H), lambda i: (0, 0)),
            pl.BlockSpec((1, H), lambda i: (0, 0)),
        ],
        out_specs=[
            pl.BlockSpec((BN, H), lambda i: (i, 0)),
            pl.BlockSpec((BN, HH), lambda i: (i, 0)),
            pl.BlockSpec((BN, HH), lambda i: (i, 0)),
        ],
        out_shape=[
            jax.ShapeDtypeStruct((N, H), _F32),
            jax.ShapeDtypeStruct((N, HH), _F32),
            jax.ShapeDtypeStruct((N, HH), _F32),
        ],
    )(h, aa, ab, uw1, ub1, uw2, ub2, mw1, mb1, mw2, mb2)


def _k3_body(h_ref, aa_ref, ab_ref, uw1_ref, ub1_ref, uw2_ref, ub2_ref,
             ow1_ref, ob1_ref, ow2_ref, ob2_ref, st_ref, preds_ref):
    pi = pl.program_id(0)
    hn = _update(h_ref[...], aa_ref[...], ab_ref[...],
                 uw1_ref, ub1_ref, uw2_ref, ub2_ref)
    t = _silu(jnp.dot(hn, ow1_ref[...], preferred_element_type=_F32)
              + ob1_ref[...])
    ao = jnp.dot(t, ow2_ref[...], preferred_element_type=_F32) + ob2_ref[...]
    ii = lax.broadcasted_iota(jnp.int32, (144, BN), 0).astype(_F32)
    pp = (lax.broadcasted_iota(jnp.int32, (144, BN), 1)
          + pi * BN).astype(_F32)
    s = st_ref[...]
    mask = ((pp >= s) & (pp < s + ii)).astype(_F32)
    part = jnp.dot(mask, ao, preferred_element_type=_F32)

    @pl.when(pi == 0)
    def _():
        preds_ref[...] = jnp.zeros_like(preds_ref)

    preds_ref[...] += part


def _k3_call(h, aa, ab, uw1, ub1, uw2, ub2, ow1, ob1, ow2_p, ob2_p, st_p):
    return pl.pallas_call(
        _k3_body,
        grid=(N // BN,),
        in_specs=[
            pl.BlockSpec((BN, H), lambda i: (i, 0)),
            pl.BlockSpec((BN, HH), lambda i: (i, 0)),
            pl.BlockSpec((BN, HH), lambda i: (i, 0)),
            pl.BlockSpec((2 * H, H), lambda i: (0, 0)),
            pl.BlockSpec((1, H), lambda i: (0, 0)),
            pl.BlockSpec((H, H), lambda i: (0, 0)),
            pl.BlockSpec((1, H), lambda i: (0, 0)),
            pl.BlockSpec((H, H), lambda i: (0, 0)),
            pl.BlockSpec((1, H), lambda i: (0, 0)),
            pl.BlockSpec((H, 8), lambda i: (0, 0)),
            pl.BlockSpec((1, 8), lambda i: (0, 0)),
            pl.BlockSpec((144, 1), lambda i: (0, 0)),
        ],
        out_specs=pl.BlockSpec((144, 8), lambda i: (0, 0)),
        out_shape=jax.ShapeDtypeStruct((144, 8), _F32),
    )(h, aa, ab, uw1, ub1, uw2, ub2, ow1, ob1, ow2_p, ob2_p, st_p)


def _edge_body(ma_hbm, mbh_hbm, da_hbm, dbh_hbm, src_hbm, tgt_hbm,
               oa_hbm, ob_hbm,
               sidx0, sidx1, sidx2, sidx3, tidx0, tidx1, tidx2, tidx3,
               grow0, grow1, drow0, drow1,
               zbuf, agg_sh, isem0, isem1, isem2, isem3,
               gsem0, gsem1, dsem0, dsem1, ssem0, ssem1):
    c = lax.axis_index("c")
    s = lax.axis_index("s")
    sidx = (sidx0, sidx1, sidx2, sidx3)
    tidx = (tidx0, tidx1, tidx2, tidx3)
    grow = (grow0, grow1)
    drow = (drow0, drow1)
    isem = (isem0, isem1, isem2, isem3)
    gsem = (gsem0, gsem1)
    dsem = (dsem0, dsem1)
    ssem = (ssem0, ssem1)
    base0 = s * EPT

    def _zrow(r, carry):
        for cc in range(HH // 16):
            zbuf[r, pl.ds(cc * 16, 16)] = jnp.zeros((16,), _F32)
        return carry

    lax.fori_loop(0, ZR, _zrow, 0)

    def _zcp(j, carry):
        ch = j * NS + s

        @pl.when(ch < NZCH)
        def _():
            pltpu.sync_copy(zbuf, agg_sh.at[pl.ds(ch * ZR, ZR)])

        return carry

    lax.fori_loop(0, ZPT, _zcp, 0)
    plsc.subcore_barrier()

    def _idx_start(k, q):
        pltpu.async_copy(src_hbm.at[pl.ds(base0 + k * CH, CH)], sidx[q],
                         isem[q])
        pltpu.async_copy(tgt_hbm.at[pl.ds(base0 + k * CH, CH)], tidx[q],
                         isem[q])

    def _idx_wait(k, q):
        pltpu.make_async_copy(src_hbm.at[pl.ds(base0 + k * CH, CH)], sidx[q],
                              isem[q]).wait()
        pltpu.make_async_copy(tgt_hbm.at[pl.ds(base0 + k * CH, CH)], tidx[q],
                              isem[q]).wait()

    def _half(m_hbm, d_hbm, o_hbm):
        def _load_start(k, b, q):
            pltpu.async_copy(m_hbm.at[sidx[q]], grow[b], gsem[b])
            pltpu.async_copy(d_hbm.at[pl.ds(base0 + k * CH, CH)], drow[b],
                             dsem[b])

        def _load_wait(k, b, q):
            pltpu.make_async_copy(m_hbm.at[sidx[q]], grow[b], gsem[b]).wait()
            pltpu.make_async_copy(d_hbm.at[pl.ds(base0 + k * CH, CH)],
                                  drow[b], dsem[b]).wait()

        def _scat_wait(b, q):
            pltpu.make_async_copy(grow[b], agg_sh.at[tidx[q]],
                                  ssem[b]).wait()

        shv = jnp.full((16,), 16, jnp.int32)
        mkv = jnp.full((16,), -65536, jnp.int32)

        def _section(k, b, q):
            @pl.when(k + 1 < NCHUNK)
            def _():
                @pl.when(k >= 1)
                def _():
                    _scat_wait(1 - b, (q + 3) % 4)
                _idx_wait(k + 1, (q + 1) % 4)
                _load_start(k + 1, 1 - b, (q + 1) % 4)

            @pl.when(k + 3 < NCHUNK)
            def _():
                _idx_start(k + 3, (q + 3) % 4)

            _load_wait(k, b, q)

            def _mul(r, cr):
                for cc in range(HH // 32):
                    di = drow[b][r, pl.ds(cc * 16, 16)]
                    dlo = lax.bitcast_convert_type(
                        lax.shift_left(di, shv), _F32)
                    dhi = lax.bitcast_convert_type(
                        lax.bitwise_and(di, mkv), _F32)
                    sl_lo = pl.ds(cc * 16, 16)
                    sl_hi = pl.ds(64 + cc * 16, 16)
                    grow[b][r, sl_lo] = grow[b][r, sl_lo] * dlo
                    grow[b][r, sl_hi] = grow[b][r, sl_hi] * dhi
                return cr

            lax.fori_loop(0, CH, _mul, 0)
            pltpu.async_copy(grow[b], agg_sh.at[tidx[q]], ssem[b], add=True)

        _idx_start(0, 0)
        _idx_start(1, 1)
        _idx_start(2, 2)
        _idx_wait(0, 0)
        _load_start(0, 0, 0)

        def _body(j, carry):
            for t in range(4):
                _section(j * 4 + t, t % 2, t)
            return carry

        lax.fori_loop(0, NCHUNK // 4, _body, 0)
        for k in range(NCHUNK - NCHUNK % 4, NCHUNK):
            _section(jnp.int32(k), k % 2, k % 4)
        _scat_wait((NCHUNK - 2) % 2, (NCHUNK - 2) % 4)
        _scat_wait((NCHUNK - 1) % 2, (NCHUNK - 1) % 4)

        plsc.subcore_barrier()

        def _wb(j, carry):
            ch = j * NS + s

            @pl.when(ch < NZCH)
            def _():
                pltpu.sync_copy(agg_sh.at[pl.ds(ch * ZR, ZR)], zbuf)
                pltpu.sync_copy(zbuf, o_hbm.at[pl.ds(ch * ZR, ZR)])

            return carry

        lax.fori_loop(0, ZPT, _wb, 0)

    @pl.when(c == 0)
    def _():
        _half(ma_hbm, da_hbm, oa_hbm)

    @pl.when(c == 1)
    def _():
        _half(mbh_hbm, dbh_hbm, ob_hbm)


def _edge_call(ma, mbh, dfa, dfb, src, tgt):
    kern = pl.kernel(
        _edge_body,
        out_type=[jax.ShapeDtypeStruct((N, HH), _F32)] * 2,
        mesh=plsc.VectorSubcoreMesh(core_axis_name="c", subcore_axis_name="s"),
        scratch_types=(
            [pltpu.VMEM((CH,), jnp.int32)] * 8
            + [
                pltpu.VMEM((CH, HH), _F32),
                pltpu.VMEM((CH, HH), _F32),
                pltpu.VMEM((CH, HH // 2), jnp.int32),
                pltpu.VMEM((CH, HH // 2), jnp.int32),
                pltpu.VMEM((ZR, HH), _F32),
                pltpu.VMEM_SHARED((N, HH), _F32),
            ]
            + [pltpu.SemaphoreType.DMA] * 10
        ),
    )
    return kern(ma, mbh, dfa, dfb, src, tgt)


def kernel(species, pcmbdf_feats, edge_index, edge_dist, n_atoms_list, embed,
           dW1, db1, dW2, db2, mW1, mb1, mW2, mb2,
           uW1, ub1, uW2, ub2, oW1, ob1, oW2, ob2):
    species = species.astype(jnp.int32)
    src = edge_index[0].astype(jnp.int32)
    tgt = edge_index[1].astype(jnp.int32)
    d2 = edge_dist[:, None].astype(_F32)
    emb_p = jnp.zeros((128, H), _F32).at[:NSP, :].set(embed.astype(_F32))

    na = n_atoms_list.astype(_F32)
    starts = jnp.concatenate([jnp.zeros((1,), _F32), jnp.cumsum(na)[:-1]])
    st_p = jnp.concatenate([starts, jnp.full((3,), 2.0 * N, _F32)])[:, None]

    ow2_p = jnp.zeros((H, 8), _F32).at[:, 0].set(oW2[:, 0])
    ob2_p = jnp.zeros((1, 8), _F32).at[0, 0].set(ob2[0])

    dfa, dfb = _dist_call(d2, dW1, db1[None], dW2, db2[None])
    h, ma, mbh = _k0_call(species.reshape(N // BN, 1, BN), emb_p,
                          mW1[0], mb1[0][None], mW2[0], mb2[0][None])
    preds = None
    for l in range(L):
        aa, ab = _edge_call(ma, mbh, dfa, dfb, src, tgt)
        if l + 1 < L:
            h, ma, mbh = _up_call(h, aa, ab,
                                  uW1[l], ub1[l][None], uW2[l], ub2[l][None],
                                  mW1[l + 1], mb1[l + 1][None],
                                  mW2[l + 1], mb2[l + 1][None])
        else:
            preds = _k3_call(h, aa, ab,
                             uW1[l], ub1[l][None], uW2[l], ub2[l][None],
                             oW1, ob1[None], ow2_p, ob2_p, st_p)
    return preds[:141, 0]
